# baseline (speedup 1.0000x reference)
import functools

import jax
import jax.numpy as jnp
from jax import lax
from jax.experimental import pallas as pl
from jax.experimental.pallas import tpu as pltpu
from jax.experimental.pallas import tpu_sc as plsc

TOPK = 32
ROW_LEN = 32768
NUM_ROWS = 32 * 32
LANES = 16
NV_ROW = ROW_LEN // LANES
GROUP = 8
NGROUP = NV_ROW // GROUP
STATIC_CAND_VREGS = 8

NEG_INF = float("-inf")
BIG_I32 = 2**31 - 1


def _tree_max(vs):
  while len(vs) > 1:
    vs = [jnp.maximum(a, b) for a, b in zip(vs[::2], vs[1::2])]
  return vs[0]


def _kernel_body(nw, in_hbm, out_hbm, row_a, row_b, cand_v, out_v,
                 sem_a, sem_b):
  rows_per_w = NUM_ROWS // nw
  info = plsc.get_sparse_core_info()
  nc = info.num_cores
  w = lax.axis_index("s") * nc + lax.axis_index("c")
  lane = lax.iota(jnp.int32, LANES)
  row0 = w * rows_per_w

  def process_row(r_local, row_v):
    ninf = jnp.full((LANES,), NEG_INF, jnp.float32)
    for k in range(STATIC_CAND_VREGS):
      cand_v[pl.ds(k * LANES, LANES)] = ninf

    P0_UNROLL = 16

    def p0(g, carry):
      a0, a1 = carry
      vs = [row_v[pl.ds((g * P0_UNROLL + k) * LANES, LANES)]
            for k in range(P0_UNROLL)]
      a0 = jnp.maximum(a0, _tree_max(vs[0::2]))
      a1 = jnp.maximum(a1, _tree_max(vs[1::2]))
      return a0, a1

    a0, a1 = lax.fori_loop(0, NV_ROW // P0_UNROLL, p0, (ninf, ninf))
    thr = jnp.min(jnp.minimum(a0, a1))

    def p1(g, ptr_vec):
      base = g * GROUP * LANES
      for k in range(GROUP):
        v = row_v[pl.ds(base + k * LANES, LANES)]
        m = v >= thr
        mi = jnp.where(m, jnp.int32(1), jnp.int32(0))
        pos = ptr_vec + plsc.cumsum(mi) - 1
        plsc.store_scatter(cand_v, [pos], v, mask=m)
        ptr_vec = ptr_vec + plsc.all_reduce_population_count(m)
      return ptr_vec

    ptr_vec = lax.fori_loop(0, NGROUP, p1, jnp.zeros((LANES,), jnp.int32))
    n_cand = jnp.max(ptr_vec)

    plsc.store_scatter(cand_v, [n_cand + lane], ninf)
    nv = lax.shift_right_logical(n_cand + LANES - 1, 4)

    def make_round(load_max, load_pos):
      def rnd(t, carry):
        v0, v1, p0_, p1_ = carry
        m_val = jnp.max(load_max())
        bp = jnp.min(load_pos(m_val))
        plsc.store_scatter(cand_v, [jnp.full((LANES,), bp, jnp.int32)],
                           ninf, mask=lane == 0)
        v0 = jnp.where(lane == t, m_val, v0)
        v1 = jnp.where(lane == t - LANES, m_val, v1)
        p0_ = jnp.where(lane == t, bp, p0_)
        p1_ = jnp.where(lane == t - LANES, bp, p1_)
        return v0, v1, p0_, p1_
      return rnd

    zf = jnp.zeros((LANES,), jnp.float32)
    zi = jnp.zeros((LANES,), jnp.int32)
    init = (zf, zf, zi, zi)

    def extract_static(_):
      def load_max():
        return _tree_max([cand_v[pl.ds(k * LANES, LANES)]
                          for k in range(STATIC_CAND_VREGS)])

      def load_pos(m_val):
        ps = []
        for k in range(STATIC_CAND_VREGS):
          v = cand_v[pl.ds(k * LANES, LANES)]
          ps.append(jnp.where(v == m_val, k * LANES + lane,
                              jnp.int32(BIG_I32)))
        while len(ps) > 1:
          ps = [jnp.minimum(a, b) for a, b in zip(ps[::2], ps[1::2])]
        return ps[0]

      return lax.fori_loop(0, TOPK, make_round(load_max, load_pos), init)

    def extract_dynamic(_):
      def load_max():
        def fmax(j, acc):
          return jnp.maximum(acc, cand_v[pl.ds(j * LANES, LANES)])
        return lax.fori_loop(0, nv, fmax, ninf)

      def load_pos(m_val):
        def fpos(j, best):
          v = cand_v[pl.ds(j * LANES, LANES)]
          return jnp.minimum(
              best, jnp.where(v == m_val, j * LANES + lane,
                              jnp.int32(BIG_I32)))
        return lax.fori_loop(0, nv, fpos,
                             jnp.full((LANES,), BIG_I32, jnp.int32))

      return lax.fori_loop(0, TOPK, make_round(load_max, load_pos), init)

    v0, v1, p0_, p1_ = lax.cond(n_cand <= STATIC_CAND_VREGS * LANES,
                                extract_static, extract_dynamic, 0)

    def rank_body(j, carry):
      r0, r1 = carry
      pj = jnp.minimum(
          jnp.min(jnp.where(lane == j, p0_, jnp.int32(BIG_I32))),
          jnp.min(jnp.where(lane == j - LANES, p1_, jnp.int32(BIG_I32))))
      r0 = r0 + jnp.where(pj < p0_, jnp.int32(1), jnp.int32(0))
      r1 = r1 + jnp.where(pj < p1_, jnp.int32(1), jnp.int32(0))
      return r0, r1

    r0, r1 = lax.fori_loop(0, TOPK, rank_body, (zi, zi))

    base = r_local * TOPK
    plsc.store_scatter(out_v, [base + r0], v0)
    plsc.store_scatter(out_v, [base + r1], v1)

  pltpu.async_copy(in_hbm.at[row0], row_a, sem_a)

  def pair_body(i, _):
    r_even = 2 * i
    pltpu.make_async_copy(in_hbm.at[row0], row_a, sem_a).wait()
    pltpu.async_copy(in_hbm.at[row0 + r_even + 1], row_b, sem_b)
    process_row(r_even, row_a)
    pltpu.make_async_copy(in_hbm.at[row0], row_b, sem_b).wait()

    @pl.when(r_even + 2 < rows_per_w)
    def _start_next():
      pltpu.async_copy(in_hbm.at[row0 + r_even + 2], row_a, sem_a)

    process_row(r_even + 1, row_b)
    return _

  lax.fori_loop(0, rows_per_w // 2, pair_body, jnp.int32(0))
  pltpu.sync_copy(out_v, out_hbm.at[pl.ds(w * rows_per_w * TOPK,
                                          rows_per_w * TOPK)])


def kernel(inputs):
  info = plsc.get_sparse_core_info()
  nw = info.num_cores * info.num_subcores
  rows_per_w = NUM_ROWS // nw
  mesh = plsc.VectorSubcoreMesh(core_axis_name="c", subcore_axis_name="s")
  k = pl.kernel(
      functools.partial(_kernel_body, nw),
      out_type=jax.ShapeDtypeStruct((NUM_ROWS * TOPK,), jnp.float32),
      mesh=mesh,
      compiler_params=pltpu.CompilerParams(needs_layout_passes=False),
      scratch_types=[
          pltpu.VMEM((ROW_LEN,), jnp.float32),
          pltpu.VMEM((ROW_LEN,), jnp.float32),
          pltpu.VMEM((ROW_LEN + LANES,), jnp.float32),
          pltpu.VMEM((rows_per_w * TOPK,), jnp.float32),
          pltpu.SemaphoreType.DMA,
          pltpu.SemaphoreType.DMA,
      ],
  )
  out = k(inputs.reshape(NUM_ROWS, ROW_LEN))
  return out.reshape(32, 32, TOPK)


# speedup vs baseline: 2.5181x; 2.5181x over previous
import functools

import jax
import jax.numpy as jnp
from jax import lax
from jax.experimental import pallas as pl
from jax.experimental.pallas import tpu as pltpu
from jax.experimental.pallas import tpu_sc as plsc

TOPK = 32
ROW_LEN = 32768
NUM_ROWS = 32 * 32
LANES = 16
NV_ROW = ROW_LEN // LANES
GROUP = 8
NGROUP = NV_ROW // GROUP
CAND_CAP = 8192
STATIC_CAND_VREGS = 8

NEG_INF = float("-inf")
BIG_I32 = 2**31 - 1


def _tree_max(vs):
  while len(vs) > 1:
    vs = [jnp.maximum(a, b) for a, b in zip(vs[::2], vs[1::2])]
  return vs[0]


def _tree_min(vs):
  while len(vs) > 1:
    vs = [jnp.minimum(a, b) for a, b in zip(vs[::2], vs[1::2])]
  return vs[0]


def _kernel_body(nw, in_hbm, out_hbm, row_a, row_b, gmax_v, dlist_v,
                 cand_v, cand_i, out_v, sem_a, sem_b):
  rows_per_w = NUM_ROWS // nw
  info = plsc.get_sparse_core_info()
  nc = info.num_cores
  w = lax.axis_index("s") * nc + lax.axis_index("c")
  lane = lax.iota(jnp.int32, LANES)
  row0 = w * rows_per_w

  def process_row(r_local, row_v):
    ninf = jnp.full((LANES,), NEG_INF, jnp.float32)
    big = jnp.full((LANES,), BIG_I32, jnp.int32)
    zi = jnp.zeros((LANES,), jnp.int32)
    zf = jnp.zeros((LANES,), jnp.float32)

    for k in range(STATIC_CAND_VREGS):
      cand_v[pl.ds(k * LANES, LANES)] = ninf

    def p0(g, carry):
      a0, a1 = carry
      vs = [row_v[pl.ds((g * 2 * GROUP + k) * LANES, LANES)]
            for k in range(2 * GROUP)]
      t0 = _tree_max(vs[:GROUP])
      t1 = _tree_max(vs[GROUP:])
      gmax_v[pl.ds((2 * g) * LANES, LANES)] = t0
      gmax_v[pl.ds((2 * g + 1) * LANES, LANES)] = t1
      return jnp.maximum(a0, t0), jnp.maximum(a1, t1)

    a0, a1 = lax.fori_loop(0, NGROUP // 2, p0, (ninf, ninf))
    thr = jnp.min(jnp.minimum(a0, a1))

    def p1a(j, dptr):
      g = gmax_v[pl.ds(j * LANES, LANES)]
      m = g >= thr
      mi = jnp.where(m, jnp.int32(1), jnp.int32(0))
      pos = dptr + plsc.cumsum(mi) - 1
      plsc.store_scatter(dlist_v, [pos], j * LANES + lane, mask=m)
      return dptr + plsc.all_reduce_population_count(m)

    dptr = lax.fori_loop(0, NGROUP, p1a, zi)
    n_dirty = jnp.max(dptr)

    def p1b(t, ptr):
      dl_idx = 2 * t + lax.shift_right_logical(lane, 3)
      valid = dl_idx < n_dirty
      e = plsc.load_gather(dlist_v, [jnp.minimum(dl_idx, NGROUP * LANES - 1)])
      e = jnp.bitwise_and(e, NGROUP * LANES - 1)
      col_base = (lax.shift_right_logical(e, 4) * (GROUP * LANES)
                  + jnp.bitwise_and(e, LANES - 1))
      elem_idx = col_base + jnp.bitwise_and(lane, 7) * LANES
      v = plsc.load_gather(row_v, [elem_idx])
      m = jnp.logical_and(v >= thr, valid)
      mi = jnp.where(m, jnp.int32(1), jnp.int32(0))
      pos = jnp.minimum(ptr + plsc.cumsum(mi) - 1, CAND_CAP)
      plsc.store_scatter(cand_v, [pos], v, mask=m)
      plsc.store_scatter(cand_i, [pos], elem_idx, mask=m)
      return ptr + plsc.all_reduce_population_count(m)

    n_pairs = lax.shift_right_logical(n_dirty + 1, 1)
    ptr = lax.fori_loop(0, n_pairs, p1b, zi)
    n_cand = jnp.max(ptr)

    plsc.store_scatter(cand_v, [jnp.minimum(n_cand, CAND_CAP) + lane], ninf)
    nv = lax.shift_right_logical(n_cand + LANES - 1, 4)

    init = (zf, zf, zi, zi)

    def extract_static(_):
      def rnd(t, carry):
        v0, v1, i0, i1 = carry
        vs = [cand_v[pl.ds(k * LANES, LANES)]
              for k in range(STATIC_CAND_VREGS)]
        m_val = jnp.max(_tree_max(vs))
        idxms = []
        for k in range(STATIC_CAND_VREGS):
          ii = cand_i[pl.ds(k * LANES, LANES)]
          idxms.append(jnp.where(vs[k] == m_val, ii, big))
        bi = jnp.min(_tree_min(idxms))
        pms = [jnp.where(idxms[k] == bi, k * LANES + lane, big)
               for k in range(STATIC_CAND_VREGS)]
        p = jnp.min(_tree_min(pms))
        plsc.store_scatter(cand_v, [jnp.full((LANES,), p, jnp.int32)],
                           ninf, mask=lane == 0)
        v0 = jnp.where(lane == t, m_val, v0)
        v1 = jnp.where(lane == t - LANES, m_val, v1)
        i0 = jnp.where(lane == t, bi, i0)
        i1 = jnp.where(lane == t - LANES, bi, i1)
        return v0, v1, i0, i1

      return lax.fori_loop(0, TOPK, rnd, init)

    def extract_dynamic(_):
      def rnd(t, carry):
        v0, v1, i0, i1 = carry

        def fmax(j, acc):
          return jnp.maximum(acc, cand_v[pl.ds(j * LANES, LANES)])

        m_val = jnp.max(lax.fori_loop(0, nv, fmax, ninf))

        def fbi(j, best):
          v = cand_v[pl.ds(j * LANES, LANES)]
          ii = cand_i[pl.ds(j * LANES, LANES)]
          return jnp.minimum(best, jnp.where(v == m_val, ii, big))

        bi = jnp.min(lax.fori_loop(0, nv, fbi, big))

        def fp(j, best):
          v = cand_v[pl.ds(j * LANES, LANES)]
          ii = cand_i[pl.ds(j * LANES, LANES)]
          hit = jnp.logical_and(v == m_val, ii == bi)
          return jnp.minimum(best, jnp.where(hit, j * LANES + lane, big))

        p = jnp.min(lax.fori_loop(0, nv, fp, big))
        plsc.store_scatter(cand_v, [jnp.full((LANES,), p, jnp.int32)],
                           ninf, mask=lane == 0)
        v0 = jnp.where(lane == t, m_val, v0)
        v1 = jnp.where(lane == t - LANES, m_val, v1)
        i0 = jnp.where(lane == t, bi, i0)
        i1 = jnp.where(lane == t - LANES, bi, i1)
        return v0, v1, i0, i1

      return lax.fori_loop(0, TOPK, rnd, init)

    def extract_fallback(_):
      plsc.store_scatter(row_v, [ROW_LEN + lane], ninf)

      def rnd(t, carry):
        v0, v1, i0, i1 = carry

        def fmax(j, acc):
          return jnp.maximum(acc, row_v[pl.ds(j * LANES, LANES)])

        m_val = jnp.max(lax.fori_loop(0, NV_ROW + 1, fmax, ninf))

        def fbi(j, best):
          v = row_v[pl.ds(j * LANES, LANES)]
          return jnp.minimum(best, jnp.where(v == m_val, j * LANES + lane,
                                             big))

        bi = jnp.min(lax.fori_loop(0, NV_ROW + 1, fbi, big))
        plsc.store_scatter(row_v, [jnp.full((LANES,), bi, jnp.int32)],
                           ninf, mask=lane == 0)
        v0 = jnp.where(lane == t, m_val, v0)
        v1 = jnp.where(lane == t - LANES, m_val, v1)
        i0 = jnp.where(lane == t, bi, i0)
        i1 = jnp.where(lane == t - LANES, bi, i1)
        return v0, v1, i0, i1

      return lax.fori_loop(0, TOPK, rnd, init)

    def extract_small(_):
      return lax.cond(n_cand <= STATIC_CAND_VREGS * LANES,
                      extract_static, extract_dynamic, 0)

    v0, v1, i0, i1 = lax.cond(n_cand <= CAND_CAP,
                              extract_small, extract_fallback, 0)

    def rank_body(j, carry):
      r0, r1 = carry
      ij = jnp.minimum(
          jnp.min(jnp.where(lane == j, i0, big)),
          jnp.min(jnp.where(lane == j - LANES, i1, big)))
      r0 = r0 + jnp.where(ij < i0, jnp.int32(1), jnp.int32(0))
      r1 = r1 + jnp.where(ij < i1, jnp.int32(1), jnp.int32(0))
      return r0, r1

    r0, r1 = lax.fori_loop(0, TOPK, rank_body, (zi, zi))

    base = r_local * TOPK
    plsc.store_scatter(out_v, [base + r0], v0)
    plsc.store_scatter(out_v, [base + r1], v1)

  pltpu.async_copy(in_hbm.at[row0], row_a.at[pl.ds(0, ROW_LEN)], sem_a)

  def pair_body(i, _):
    r_even = 2 * i
    pltpu.make_async_copy(in_hbm.at[row0], row_a.at[pl.ds(0, ROW_LEN)],
                          sem_a).wait()
    pltpu.async_copy(in_hbm.at[row0 + r_even + 1],
                     row_b.at[pl.ds(0, ROW_LEN)], sem_b)
    process_row(r_even, row_a)
    pltpu.make_async_copy(in_hbm.at[row0], row_b.at[pl.ds(0, ROW_LEN)],
                          sem_b).wait()

    @pl.when(r_even + 2 < rows_per_w)
    def _start_next():
      pltpu.async_copy(in_hbm.at[row0 + r_even + 2],
                       row_a.at[pl.ds(0, ROW_LEN)], sem_a)

    process_row(r_even + 1, row_b)
    return _

  lax.fori_loop(0, rows_per_w // 2, pair_body, jnp.int32(0))
  pltpu.sync_copy(out_v, out_hbm.at[pl.ds(w * rows_per_w * TOPK,
                                          rows_per_w * TOPK)])


def kernel(inputs):
  info = plsc.get_sparse_core_info()
  nw = info.num_cores * info.num_subcores
  rows_per_w = NUM_ROWS // nw
  mesh = plsc.VectorSubcoreMesh(core_axis_name="c", subcore_axis_name="s")
  k = pl.kernel(
      functools.partial(_kernel_body, nw),
      out_type=jax.ShapeDtypeStruct((NUM_ROWS * TOPK,), jnp.float32),
      mesh=mesh,
      compiler_params=pltpu.CompilerParams(needs_layout_passes=False),
      scratch_types=[
          pltpu.VMEM((ROW_LEN + LANES,), jnp.float32),
          pltpu.VMEM((ROW_LEN + LANES,), jnp.float32),
          pltpu.VMEM((NGROUP * LANES,), jnp.float32),
          pltpu.VMEM((NGROUP * LANES,), jnp.int32),
          pltpu.VMEM((CAND_CAP + LANES,), jnp.float32),
          pltpu.VMEM((CAND_CAP + LANES,), jnp.int32),
          pltpu.VMEM((rows_per_w * TOPK,), jnp.float32),
          pltpu.SemaphoreType.DMA,
          pltpu.SemaphoreType.DMA,
      ],
  )
  out = k(inputs.reshape(NUM_ROWS, ROW_LEN))
  return out.reshape(32, 32, TOPK)
